# baseline (speedup 1.0000x reference)
import math

import jax
import jax.numpy as jnp
from jax import lax
from jax.experimental import pallas as pl
from jax.experimental.pallas import tpu as pltpu

TEMPERATURE = 0.5
SPARSITY_THRESHOLD = 0.01
EDGE_WEIGHT = 0.1
LN_EPS = 1e-5


def _layer_norm(x, w, b):
    mean = jnp.mean(x, axis=-1, keepdims=True)
    m2 = jnp.mean(x * x, axis=-1, keepdims=True)
    var = m2 - mean * mean
    return (x - mean) * lax.rsqrt(var + LN_EPS) * w + b


def _gnn_chain(e, w1_ref, w2_ref, w3_ref, lnw, lnb, wout_ref, bout):
    BB, S, D = e.shape
    H = w1_ref.shape[1]
    V = wout_ref.shape[1]
    N = BB * S
    bf16 = jnp.bfloat16

    nsq = jnp.sum(e * e, axis=-1, keepdims=True)
    en = e * lax.rsqrt(jnp.maximum(nsq, 1e-12))

    n2 = jnp.sum(en * en, axis=-1, keepdims=True)
    dots = jnp.einsum('bsd,btd->bst', en, en,
                      preferred_element_type=jnp.float32)
    n2b = jnp.broadcast_to(n2, (BB, S, S))
    sq = n2b + jnp.transpose(n2b, (0, 2, 1)) - 2.0 * dots
    dist = jnp.sqrt(jnp.maximum(sq, 1e-12))
    row = lax.broadcasted_iota(jnp.int32, (BB, S, S), 1)
    col = lax.broadcasted_iota(jnp.int32, (BB, S, S), 2)
    off_diag = row != col
    dist = jnp.where(off_diag, dist, 0.0)
    scaled = dist * (-1.0 / TEMPERATURE)
    m = jnp.max(scaled, axis=-1, keepdims=True)
    p = jnp.exp(scaled - m)
    denom = jnp.sum(p, axis=-1, keepdims=True)
    attn = p * pl.reciprocal(denom, approx=True)

    A = jnp.where((attn > SPARSITY_THRESHOLD) & off_diag,
                  attn * EDGE_WEIGHT, 0.0)
    At = jnp.transpose(A, (0, 2, 1)).astype(bf16)

    def message_pass(x_flat, w_ref):
        h = jnp.dot(x_flat.astype(bf16), w_ref[...],
                    preferred_element_type=jnp.float32)
        h3 = h.reshape(BB, S, h.shape[-1]).astype(bf16)
        agg = jnp.einsum('bds,bsh->bdh', At, h3,
                         preferred_element_type=jnp.float32)
        return agg.reshape(N, h.shape[-1])

    x = en.reshape(N, D)
    x = _layer_norm(message_pass(x, w1_ref), lnw, lnb)
    x = _layer_norm(x + message_pass(x, w2_ref), lnw, lnb)
    x = _layer_norm(x + message_pass(x, w3_ref), lnw, lnb)

    logits = jnp.dot(x.astype(bf16), wout_ref[...],
                     preferred_element_type=jnp.float32) + bout
    return logits.reshape(BB, S, V)


def _fused_gnn_kernel(emb_ref, w1_ref, w2_ref, w3_ref, lnw_ref, lnb_ref,
                      wout_ref, bout_ref, out_ref):
    BB2, S, D = emb_ref.shape
    BB = BB2 // 2
    lnw = lnw_ref[...]
    lnb = lnb_ref[...]
    bout = bout_ref[...]
    e = emb_ref[...]
    out_ref[:BB] = _gnn_chain(e[:BB], w1_ref, w2_ref, w3_ref,
                              lnw, lnb, wout_ref, bout)
    out_ref[BB:] = _gnn_chain(e[BB:], w1_ref, w2_ref, w3_ref,
                              lnw, lnb, wout_ref, bout)


def kernel(tokens, token_embedding, w1, w2, w3, ln_w, ln_b, w_out, b_out):
    B, S = tokens.shape
    V, D = token_embedding.shape
    H = w1.shape[1]
    Vout = w_out.shape[1]
    max_seq_len = 512
    num_batch_blocks = 32
    BB2 = B // num_batch_blocks

    pos = jnp.arange(S, dtype=jnp.float32)
    thetas = 2.0 * math.pi * (pos / max_seq_len)
    rs = 0.3 + 0.6 * (pos / max(1, max_seq_len - 1))
    spiral = jnp.stack([rs * jnp.cos(thetas), rs * jnp.sin(thetas)], axis=-1)
    spiral = jnp.broadcast_to(spiral[None], (B, S, 2))
    token_embs = token_embedding[tokens][:, :, : D - 2]
    emb = jnp.concatenate([spiral, token_embs], axis=-1)

    w1b = w1.astype(jnp.bfloat16)
    w2b = w2.astype(jnp.bfloat16)
    w3b = w3.astype(jnp.bfloat16)
    woutb = w_out.astype(jnp.bfloat16)

    return pl.pallas_call(
        _fused_gnn_kernel,
        out_shape=jax.ShapeDtypeStruct((B, S, Vout), jnp.float32),
        grid_spec=pltpu.PrefetchScalarGridSpec(
            num_scalar_prefetch=0,
            grid=(num_batch_blocks,),
            in_specs=[
                pl.BlockSpec((BB2, S, D), lambda b: (b, 0, 0)),
                pl.BlockSpec((D, H), lambda b: (0, 0)),
                pl.BlockSpec((H, H), lambda b: (0, 0)),
                pl.BlockSpec((H, H), lambda b: (0, 0)),
                pl.BlockSpec((1, H), lambda b: (0, 0)),
                pl.BlockSpec((1, H), lambda b: (0, 0)),
                pl.BlockSpec((H, Vout), lambda b: (0, 0)),
                pl.BlockSpec((1, Vout), lambda b: (0, 0)),
            ],
            out_specs=pl.BlockSpec((BB2, S, Vout), lambda b: (b, 0, 0)),
        ),
        compiler_params=pltpu.CompilerParams(dimension_semantics=("parallel",)),
    )(emb, w1b, w2b, w3b, ln_w, ln_b, woutb, b_out)


# speedup vs baseline: 1.0274x; 1.0274x over previous
import math

import jax
import jax.numpy as jnp
from jax import lax
from jax.experimental import pallas as pl
from jax.experimental.pallas import tpu as pltpu

TEMPERATURE = 0.5
SPARSITY_THRESHOLD = 0.01
EDGE_WEIGHT = 0.1
LN_EPS = 1e-5


def _layer_norm(x, w, b):
    mean = jnp.mean(x, axis=-1, keepdims=True)
    m2 = jnp.mean(x * x, axis=-1, keepdims=True)
    var = m2 - mean * mean
    return (x - mean) * lax.rsqrt(var + LN_EPS) * w + b


def _gnn_chain(e, w1_ref, w2_ref, w3_ref, lnw, lnb, wout_ref, bout):
    BB, S, D = e.shape
    H = w1_ref.shape[1]
    V = wout_ref.shape[1]
    N = BB * S
    bf16 = jnp.bfloat16

    nsq = jnp.sum(e * e, axis=-1, keepdims=True)
    en = e * lax.rsqrt(jnp.maximum(nsq, 1e-12))

    n2 = jnp.sum(en * en, axis=-1, keepdims=True)
    dots = jnp.einsum('bsd,btd->bst', en, en,
                      preferred_element_type=jnp.float32)
    n2b = jnp.broadcast_to(n2, (BB, S, S))
    sq = n2b + jnp.transpose(n2b, (0, 2, 1)) - 2.0 * dots
    dist = jnp.sqrt(jnp.maximum(sq, 1e-12))
    row = lax.broadcasted_iota(jnp.int32, (BB, S, S), 1)
    col = lax.broadcasted_iota(jnp.int32, (BB, S, S), 2)
    off_diag = row != col
    dist = jnp.where(off_diag, dist, 0.0)
    scaled = dist * (-1.0 / TEMPERATURE)
    m = jnp.max(scaled, axis=-1, keepdims=True)
    p = jnp.exp(scaled - m)
    denom = jnp.sum(p, axis=-1, keepdims=True)
    attn = p * pl.reciprocal(denom, approx=True)

    A = jnp.where((attn > SPARSITY_THRESHOLD) & off_diag,
                  attn * EDGE_WEIGHT, 0.0)
    At = jnp.transpose(A, (0, 2, 1)).astype(bf16)

    def message_pass(x_flat, w_ref):
        h = jnp.dot(x_flat.astype(bf16), w_ref[...],
                    preferred_element_type=jnp.float32)
        h3 = h.reshape(BB, S, h.shape[-1]).astype(bf16)
        agg = jnp.einsum('bds,bsh->bdh', At, h3,
                         preferred_element_type=jnp.float32)
        return agg.reshape(N, h.shape[-1])

    x = en.reshape(N, D)
    x = _layer_norm(message_pass(x, w1_ref), lnw, lnb)
    x = _layer_norm(x + message_pass(x, w2_ref), lnw, lnb)
    x = _layer_norm(x + message_pass(x, w3_ref), lnw, lnb)

    logits = jnp.dot(x.astype(bf16), wout_ref[...],
                     preferred_element_type=jnp.float32) + bout
    return logits.reshape(BB, S, V)


def _fused_gnn_kernel(emb_ref, w1_ref, w2_ref, w3_ref, lnw_ref, lnb_ref,
                      wout_ref, bout_ref, out_ref):
    lnw = lnw_ref[...]
    lnb = lnb_ref[...]
    bout = bout_ref[...]
    out_ref[...] = _gnn_chain(emb_ref[...], w1_ref, w2_ref, w3_ref,
                              lnw, lnb, wout_ref, bout)


def kernel(tokens, token_embedding, w1, w2, w3, ln_w, ln_b, w_out, b_out):
    B, S = tokens.shape
    V, D = token_embedding.shape
    H = w1.shape[1]
    Vout = w_out.shape[1]
    max_seq_len = 512
    num_batch_blocks = 32
    BB2 = B // num_batch_blocks

    pos = jnp.arange(S, dtype=jnp.float32)
    thetas = 2.0 * math.pi * (pos / max_seq_len)
    rs = 0.3 + 0.6 * (pos / max(1, max_seq_len - 1))
    spiral = jnp.stack([rs * jnp.cos(thetas), rs * jnp.sin(thetas)], axis=-1)
    spiral = jnp.broadcast_to(spiral[None], (B, S, 2))
    token_embs = token_embedding[tokens][:, :, : D - 2]
    emb = jnp.concatenate([spiral, token_embs], axis=-1)

    w1b = w1.astype(jnp.bfloat16)
    w2b = w2.astype(jnp.bfloat16)
    w3b = w3.astype(jnp.bfloat16)
    woutb = w_out.astype(jnp.bfloat16)

    return pl.pallas_call(
        _fused_gnn_kernel,
        out_shape=jax.ShapeDtypeStruct((B, S, Vout), jnp.float32),
        grid_spec=pltpu.PrefetchScalarGridSpec(
            num_scalar_prefetch=0,
            grid=(num_batch_blocks,),
            in_specs=[
                pl.BlockSpec((BB2, S, D), lambda b: (b, 0, 0)),
                pl.BlockSpec((D, H), lambda b: (0, 0)),
                pl.BlockSpec((H, H), lambda b: (0, 0)),
                pl.BlockSpec((H, H), lambda b: (0, 0)),
                pl.BlockSpec((1, H), lambda b: (0, 0)),
                pl.BlockSpec((1, H), lambda b: (0, 0)),
                pl.BlockSpec((H, Vout), lambda b: (0, 0)),
                pl.BlockSpec((1, Vout), lambda b: (0, 0)),
            ],
            out_specs=pl.BlockSpec((BB2, S, Vout), lambda b: (b, 0, 0)),
        ),
        compiler_params=pltpu.CompilerParams(dimension_semantics=("parallel",)),
    )(emb, w1b, w2b, w3b, ln_w, ln_b, woutb, b_out)
